# baseline (speedup 1.0000x reference)
import functools

import jax
import jax.numpy as jnp
from jax import lax
from jax.experimental import pallas as pl
from jax.experimental.pallas import tpu as pltpu
from jax.experimental.pallas import tpu_sc as plsc

N = 10000
NP = 10240
L = 128
NC = 2
NS = 16
NW = NC * NS
RPT = NP // NS
RB = 4
RC = RPT // 2
TF = 56
TS = 104
TMX = max(TF, TS)
BLK = 2048

_P = jax.lax.Precision.HIGHEST


def _deg_call(dst3, zvec):
    TPW = dst3.shape[1]
    mesh = plsc.VectorSubcoreMesh(core_axis_name="c", subcore_axis_name="s")

    @functools.partial(
        pl.kernel,
        out_type=jax.ShapeDtypeStruct((NC, NP), jnp.float32),
        mesh=mesh,
        compiler_params=pltpu.CompilerParams(use_tc_tiling_on_sc=False),
        scratch_types=[
            pltpu.VMEM((TPW, L), jnp.int32),
            pltpu.VMEM((L,), jnp.float32),
            pltpu.VMEM((RPT,), jnp.float32),
            pltpu.VMEM_SHARED((NP,), jnp.float32),
            pltpu.SemaphoreType.DMA,
        ],
    )
    def deg_kernel(dst_hbm, zvec_hbm, out_hbm, dst_v, ones_v, rb_v, acc_sh, sem):
        c = lax.axis_index("c")
        s = lax.axis_index("s")
        w = c * NS + s
        LAG = 8
        pltpu.sync_copy(zvec_hbm, rb_v)
        pltpu.sync_copy(rb_v, acc_sh.at[pl.ds(s * RPT, RPT)])
        for k in range(L // 16):
            ones_v[pl.ds(k * 16, 16)] = jnp.full((16,), 1.0, jnp.float32)
        pltpu.sync_copy(dst_hbm.at[w], dst_v)
        plsc.subcore_barrier()

        def body(j, carry):
            @pl.when(j >= LAG)
            def _():
                pltpu.make_async_copy(ones_v, acc_sh.at[dst_v.at[0]], sem).wait()

            pltpu.async_copy(ones_v, acc_sh.at[dst_v.at[j]], sem, add=True)
            return carry

        lax.fori_loop(0, TPW, body, 0)
        for _ in range(min(LAG, TPW)):
            pltpu.make_async_copy(ones_v, acc_sh.at[dst_v.at[0]], sem).wait()
        plsc.subcore_barrier()
        pltpu.sync_copy(acc_sh.at[pl.ds(s * RPT, RPT)], rb_v)
        pltpu.sync_copy(rb_v, out_hbm.at[c, pl.ds(s * RPT, RPT)])

    return deg_kernel(dst3, zvec)


def _agg_call(hs, src_f, dst_f, zrows):
    D = hs.shape[1]
    mesh = plsc.VectorSubcoreMesh(core_axis_name="c", subcore_axis_name="s")

    @functools.partial(
        pl.kernel,
        out_type=jax.ShapeDtypeStruct((NC, NP, D), jnp.float32),
        mesh=mesh,
        compiler_params=pltpu.CompilerParams(use_tc_tiling_on_sc=False),
        scratch_types=[
            pltpu.VMEM((TMX, L), jnp.int32),
            pltpu.VMEM((TMX, L), jnp.int32),
            pltpu.VMEM((RB * L, D), jnp.float32),
            pltpu.VMEM((RC, D), jnp.float32),
            pltpu.VMEM_SHARED((NP, D), jnp.float32),
            pltpu.SemaphoreType.DMA,
            pltpu.SemaphoreType.DMA,
        ],
    )
    def agg_kernel(hs_hbm, src_hbm, dst_hbm, zrows_hbm, out_hbm,
                   src_v, dst_v, rows_v, big_v, acc_sh, gsem, ssem):
        c = lax.axis_index("c")
        s = lax.axis_index("s")
        row0 = jnp.where(c == 0, s * TF, NS * TF + s * TS)
        tpw = jnp.where(c == 0, TF, TS)
        pltpu.sync_copy(zrows_hbm, big_v)
        for h in range(RPT // RC):
            pltpu.sync_copy(big_v, acc_sh.at[pl.ds(s * RPT + h * RC, RC)])
        pltpu.sync_copy(src_hbm.at[pl.ds(row0, TMX)], src_v)
        pltpu.sync_copy(dst_hbm.at[pl.ds(row0, TMX)], dst_v)
        plsc.subcore_barrier()

        for b in range(RB - 1):
            pltpu.async_copy(hs_hbm.at[src_v.at[b]],
                             rows_v.at[pl.ds(b * L, L)], gsem)

        def body(j, carry):
            slot = lax.rem(j, RB) * L
            nxt = j + RB - 1
            nslot = lax.rem(nxt, RB) * L

            @pl.when(j >= 1)
            def _():
                pltpu.make_async_copy(rows_v.at[pl.ds(0, L)],
                                      acc_sh.at[dst_v.at[0]], ssem).wait()

            @pl.when(nxt < tpw)
            def _():
                pltpu.async_copy(hs_hbm.at[src_v.at[nxt]],
                                 rows_v.at[pl.ds(nslot, L)], gsem)

            pltpu.make_async_copy(hs_hbm.at[src_v.at[j]],
                                  rows_v.at[pl.ds(slot, L)], gsem).wait()
            pltpu.async_copy(rows_v.at[pl.ds(slot, L)],
                             acc_sh.at[dst_v.at[j]], ssem, add=True)
            return carry

        lax.fori_loop(0, tpw, body, 0)
        pltpu.make_async_copy(rows_v.at[pl.ds(0, L)],
                              acc_sh.at[dst_v.at[0]], ssem).wait()
        plsc.subcore_barrier()
        for h in range(RPT // RC):
            pltpu.sync_copy(acc_sh.at[pl.ds(s * RPT + h * RC, RC)], big_v)
            pltpu.sync_copy(big_v, out_hbm.at[c, pl.ds(s * RPT + h * RC, RC)])

    return agg_kernel(hs, src_f, dst_f, zrows)


def _hs_body(x_ref, w1_ref, w2_ref, degt_ref, out_ref):
    wc = lax.dot_general(w2_ref[...], w1_ref[...], (((1,), (0,)), ((), ())),
                         precision=_P, preferred_element_type=jnp.float32)
    h = lax.dot_general(x_ref[...], wc, (((1,), (1,)), ((), ())),
                        precision=_P, preferred_element_type=jnp.float32)
    deg = jnp.sum(degt_ref[...], axis=1, keepdims=True) + 1.0
    out_ref[...] = h * lax.rsqrt(deg)


def _final_body(p0_ref, p1_ref, hs_ref, degt_ref, w2_ref, b1_ref, b2_ref, out_ref):
    deg = jnp.sum(degt_ref[...], axis=1, keepdims=True) + 1.0
    dinv = lax.rsqrt(deg)
    bc = lax.dot_general(b1_ref[...], w2_ref[...], (((1,), (1,)), ((), ())),
                         precision=_P, preferred_element_type=jnp.float32)
    out_ref[...] = dinv * (p0_ref[...] + p1_ref[...] + hs_ref[...]) + (bc + b2_ref[...])


def kernel(x, edge_index, W1, b1, W2, b2):
    D = W2.shape[0]
    NF = x.shape[1]
    E = edge_index.shape[1]
    EPW = ((E + NW * L - 1) // (NW * L)) * L
    EP = EPW * NW
    TPW = EPW // L
    src = jnp.concatenate([edge_index[0], jnp.zeros((EP - E,), jnp.int32)])
    pad_dst = N + jnp.arange(EP - E, dtype=jnp.int32) % (NP - N)
    dst = jnp.concatenate([edge_index[1], pad_dst])
    dst3 = dst.reshape(NW, TPW, L)
    FLAT = NS * TF + (NS - 1) * TS + TMX
    extra = FLAT * L - EP
    src_f = jnp.concatenate([src, jnp.zeros((extra,), jnp.int32)]).reshape(FLAT, L)
    dst_f = jnp.concatenate([dst, jnp.full((extra,), N, jnp.int32)]).reshape(FLAT, L)

    zvec = jnp.zeros((RPT,), jnp.float32)
    zrows = jnp.zeros((RC, D), jnp.float32)

    deg2 = _deg_call(dst3, zvec)
    deg_t = deg2.T

    xp = jnp.pad(x, ((0, NP - N), (0, 0)))

    hs = pl.pallas_call(
        _hs_body,
        grid=(NP // BLK,),
        in_specs=[
            pl.BlockSpec((BLK, NF), lambda i: (i, 0)),
            pl.BlockSpec(W1.shape, lambda i: (0, 0)),
            pl.BlockSpec(W2.shape, lambda i: (0, 0)),
            pl.BlockSpec((BLK, NC), lambda i: (i, 0)),
        ],
        out_specs=pl.BlockSpec((BLK, D), lambda i: (i, 0)),
        out_shape=jax.ShapeDtypeStruct((NP, D), jnp.float32),
    )(xp, W1, W2, deg_t)

    parts = _agg_call(hs, src_f, dst_f, zrows)

    out = pl.pallas_call(
        _final_body,
        grid=(NP // BLK,),
        in_specs=[
            pl.BlockSpec((BLK, D), lambda i: (i, 0)),
            pl.BlockSpec((BLK, D), lambda i: (i, 0)),
            pl.BlockSpec((BLK, D), lambda i: (i, 0)),
            pl.BlockSpec((BLK, NC), lambda i: (i, 0)),
            pl.BlockSpec(W2.shape, lambda i: (0, 0)),
            pl.BlockSpec((1, W1.shape[1]), lambda i: (0, 0)),
            pl.BlockSpec((1, D), lambda i: (0, 0)),
        ],
        out_specs=pl.BlockSpec((BLK, D), lambda i: (i, 0)),
        out_shape=jax.ShapeDtypeStruct((NP, D), jnp.float32),
    )(parts[0], parts[1], hs, deg_t, W2, b1.reshape(1, -1), b2.reshape(1, -1))

    return out[:N]


# speedup vs baseline: 2.4853x; 2.4853x over previous
import functools

import jax
import jax.numpy as jnp
from jax import lax
from jax.experimental import pallas as pl
from jax.experimental.pallas import tpu as pltpu
from jax.experimental.pallas import tpu_sc as plsc

N = 10000
NP = 10240
L = 128
NC = 2
NS = 16
NW = NC * NS
RPT = NP // NS
RB = 4
RC = RPT // 2
BLK = 2048

_P = jax.lax.Precision.HIGHEST


def _deg_call(dst3, zvec):
    TPW = dst3.shape[1]
    mesh = plsc.VectorSubcoreMesh(core_axis_name="c", subcore_axis_name="s")

    @functools.partial(
        pl.kernel,
        out_type=jax.ShapeDtypeStruct((NC, NP), jnp.float32),
        mesh=mesh,
        compiler_params=pltpu.CompilerParams(use_tc_tiling_on_sc=False),
        scratch_types=[
            pltpu.VMEM((TPW, L), jnp.int32),
            pltpu.VMEM((L,), jnp.float32),
            pltpu.VMEM((RPT,), jnp.float32),
            pltpu.VMEM_SHARED((NP,), jnp.float32),
            pltpu.SemaphoreType.DMA,
        ],
    )
    def deg_kernel(dst_hbm, zvec_hbm, out_hbm, dst_v, ones_v, rb_v, acc_sh, sem):
        c = lax.axis_index("c")
        s = lax.axis_index("s")
        w = c * NS + s
        LAG = 8
        pltpu.sync_copy(zvec_hbm, rb_v)
        pltpu.sync_copy(rb_v, acc_sh.at[pl.ds(s * RPT, RPT)])
        for k in range(L // 16):
            ones_v[pl.ds(k * 16, 16)] = jnp.full((16,), 1.0, jnp.float32)
        pltpu.sync_copy(dst_hbm.at[w], dst_v)
        plsc.subcore_barrier()

        def body(j, carry):
            @pl.when(j >= LAG)
            def _():
                pltpu.make_async_copy(ones_v, acc_sh.at[dst_v.at[0]], sem).wait()

            pltpu.async_copy(ones_v, acc_sh.at[dst_v.at[j]], sem, add=True)
            return carry

        lax.fori_loop(0, TPW, body, 0)
        for _ in range(min(LAG, TPW)):
            pltpu.make_async_copy(ones_v, acc_sh.at[dst_v.at[0]], sem).wait()
        plsc.subcore_barrier()
        pltpu.sync_copy(acc_sh.at[pl.ds(s * RPT, RPT)], rb_v)
        pltpu.sync_copy(rb_v, out_hbm.at[c, pl.ds(s * RPT, RPT)])

    return deg_kernel(dst3, zvec)


def _agg_call(hs, src3, dst3, zrows):
    D = hs.shape[1]
    TPW = src3.shape[1]
    mesh = plsc.VectorSubcoreMesh(core_axis_name="c", subcore_axis_name="s")

    @functools.partial(
        pl.kernel,
        out_type=jax.ShapeDtypeStruct((NC, NP, D), jnp.float32),
        mesh=mesh,
        compiler_params=pltpu.CompilerParams(use_tc_tiling_on_sc=False),
        scratch_types=[
            pltpu.VMEM((TPW, L), jnp.int32),
            pltpu.VMEM((TPW, L), jnp.int32),
            pltpu.VMEM((RB * L, D), jnp.float32),
            pltpu.VMEM((RC, D), jnp.float32),
            pltpu.VMEM_SHARED((NP, D), jnp.float32),
            pltpu.SemaphoreType.DMA,
            pltpu.SemaphoreType.DMA,
        ],
    )
    def agg_kernel(hs_hbm, src_hbm, dst_hbm, zrows_hbm, out_hbm,
                   src_v, dst_v, rows_v, big_v, acc_sh, gsem, ssem):
        c = lax.axis_index("c")
        s = lax.axis_index("s")
        w = c * NS + s
        tpw = TPW
        pltpu.sync_copy(zrows_hbm, big_v)
        for h in range(RPT // RC):
            pltpu.sync_copy(big_v, acc_sh.at[pl.ds(s * RPT + h * RC, RC)])
        pltpu.sync_copy(src_hbm.at[w], src_v)
        pltpu.sync_copy(dst_hbm.at[w], dst_v)
        plsc.subcore_barrier()

        for b in range(RB - 1):
            pltpu.async_copy(hs_hbm.at[src_v.at[b]],
                             rows_v.at[pl.ds(b * L, L)], gsem)

        def body(j, carry):
            slot = lax.rem(j, RB) * L
            nxt = j + RB - 1
            nslot = lax.rem(nxt, RB) * L

            @pl.when(j >= 1)
            def _():
                pltpu.make_async_copy(rows_v.at[pl.ds(0, L)],
                                      acc_sh.at[dst_v.at[0]], ssem).wait()

            @pl.when(nxt < tpw)
            def _():
                pltpu.async_copy(hs_hbm.at[src_v.at[nxt]],
                                 rows_v.at[pl.ds(nslot, L)], gsem)

            pltpu.make_async_copy(hs_hbm.at[src_v.at[j]],
                                  rows_v.at[pl.ds(slot, L)], gsem).wait()
            pltpu.async_copy(rows_v.at[pl.ds(slot, L)],
                             acc_sh.at[dst_v.at[j]], ssem, add=True)
            return carry

        lax.fori_loop(0, tpw, body, 0)
        pltpu.make_async_copy(rows_v.at[pl.ds(0, L)],
                              acc_sh.at[dst_v.at[0]], ssem).wait()
        plsc.subcore_barrier()
        for h in range(RPT // RC):
            pltpu.sync_copy(acc_sh.at[pl.ds(s * RPT + h * RC, RC)], big_v)
            pltpu.sync_copy(big_v, out_hbm.at[c, pl.ds(s * RPT + h * RC, RC)])

    return agg_kernel(hs, src3, dst3, zrows)


def _hs_body(x_ref, w1_ref, w2_ref, degt_ref, out_ref):
    wc = lax.dot_general(w2_ref[...], w1_ref[...], (((1,), (0,)), ((), ())),
                         precision=_P, preferred_element_type=jnp.float32)
    h = lax.dot_general(x_ref[...], wc, (((1,), (1,)), ((), ())),
                        precision=_P, preferred_element_type=jnp.float32)
    deg = jnp.sum(degt_ref[...], axis=1, keepdims=True) + 1.0
    out_ref[...] = h * lax.rsqrt(deg)


def _final_body(p0_ref, p1_ref, hs_ref, degt_ref, w2_ref, b1_ref, b2_ref, out_ref):
    deg = jnp.sum(degt_ref[...], axis=1, keepdims=True) + 1.0
    dinv = lax.rsqrt(deg)
    bc = lax.dot_general(b1_ref[...], w2_ref[...], (((1,), (1,)), ((), ())),
                         precision=_P, preferred_element_type=jnp.float32)
    out_ref[...] = dinv * (p0_ref[...] + p1_ref[...] + hs_ref[...]) + (bc + b2_ref[...])


def kernel(x, edge_index, W1, b1, W2, b2):
    D = W2.shape[0]
    NF = x.shape[1]
    E = edge_index.shape[1]
    EPW = ((E + NW * L - 1) // (NW * L)) * L
    EP = EPW * NW
    TPW = EPW // L
    EW = E // NW
    PW = EPW - EW
    kk = jnp.arange(PW, dtype=jnp.int32)
    ww = jnp.arange(NW, dtype=jnp.int32)[:, None]
    pad_src = N + (ww * 7 + kk[None, :]) % (NP - N)
    pad_dst_agg = (ww * 317 + kk[None, :] * 41) % N
    pad_dst_deg = N + (ww * 7 + kk[None, :]) % (NP - N)
    src3 = jnp.concatenate(
        [edge_index[0].reshape(NW, EW), pad_src], axis=1).reshape(NW, TPW, L)
    dst3 = jnp.concatenate(
        [edge_index[1].reshape(NW, EW), pad_dst_agg], axis=1).reshape(NW, TPW, L)
    dst3_deg = jnp.concatenate(
        [edge_index[1].reshape(NW, EW), pad_dst_deg], axis=1).reshape(NW, TPW, L)

    zvec = jnp.zeros((RPT,), jnp.float32)
    zrows = jnp.zeros((RC, D), jnp.float32)

    deg2 = _deg_call(dst3_deg, zvec)
    deg_t = deg2.T

    xp = jnp.pad(x, ((0, NP - N), (0, 0)))

    hs = pl.pallas_call(
        _hs_body,
        grid=(NP // BLK,),
        in_specs=[
            pl.BlockSpec((BLK, NF), lambda i: (i, 0)),
            pl.BlockSpec(W1.shape, lambda i: (0, 0)),
            pl.BlockSpec(W2.shape, lambda i: (0, 0)),
            pl.BlockSpec((BLK, NC), lambda i: (i, 0)),
        ],
        out_specs=pl.BlockSpec((BLK, D), lambda i: (i, 0)),
        out_shape=jax.ShapeDtypeStruct((NP, D), jnp.float32),
    )(xp, W1, W2, deg_t)

    parts = _agg_call(hs, src3, dst3, zrows)

    out = pl.pallas_call(
        _final_body,
        grid=(NP // BLK,),
        in_specs=[
            pl.BlockSpec((BLK, D), lambda i: (i, 0)),
            pl.BlockSpec((BLK, D), lambda i: (i, 0)),
            pl.BlockSpec((BLK, D), lambda i: (i, 0)),
            pl.BlockSpec((BLK, NC), lambda i: (i, 0)),
            pl.BlockSpec(W2.shape, lambda i: (0, 0)),
            pl.BlockSpec((1, W1.shape[1]), lambda i: (0, 0)),
            pl.BlockSpec((1, D), lambda i: (0, 0)),
        ],
        out_specs=pl.BlockSpec((BLK, D), lambda i: (i, 0)),
        out_shape=jax.ShapeDtypeStruct((NP, D), jnp.float32),
    )(parts[0], parts[1], hs, deg_t, W2, b1.reshape(1, -1), b2.reshape(1, -1))

    return out[:N]


# speedup vs baseline: 2.5724x; 1.0351x over previous
import functools

import jax
import jax.numpy as jnp
from jax import lax
from jax.experimental import pallas as pl
from jax.experimental.pallas import tpu as pltpu
from jax.experimental.pallas import tpu_sc as plsc

N = 10000
NP = 10240
L = 128
NC = 2
NS = 16
NW = NC * NS
RPT = NP // NS
RB = 4
RC = RPT // 2
BLK = 2048

_P = jax.lax.Precision.HIGHEST


def _deg_call(dst3, zvec):
    TPW = dst3.shape[1]
    mesh = plsc.VectorSubcoreMesh(core_axis_name="c", subcore_axis_name="s")

    @functools.partial(
        pl.kernel,
        out_type=jax.ShapeDtypeStruct((NC, NP), jnp.float32),
        mesh=mesh,
        compiler_params=pltpu.CompilerParams(use_tc_tiling_on_sc=False),
        scratch_types=[
            pltpu.VMEM((TPW, L), jnp.int32),
            pltpu.VMEM((L,), jnp.float32),
            pltpu.VMEM((RPT,), jnp.float32),
            pltpu.VMEM_SHARED((NP,), jnp.float32),
            pltpu.SemaphoreType.DMA,
        ],
    )
    def deg_kernel(dst_hbm, zvec_hbm, out_hbm, dst_v, ones_v, rb_v, acc_sh, sem):
        c = lax.axis_index("c")
        s = lax.axis_index("s")
        w = c * NS + s
        LAG = 8
        pltpu.sync_copy(zvec_hbm, rb_v)
        pltpu.sync_copy(rb_v, acc_sh.at[pl.ds(s * RPT, RPT)])
        for k in range(L // 16):
            ones_v[pl.ds(k * 16, 16)] = jnp.full((16,), 1.0, jnp.float32)
        pltpu.sync_copy(dst_hbm.at[w], dst_v)
        plsc.subcore_barrier()

        def body(j, carry):
            @pl.when(j >= LAG)
            def _():
                pltpu.make_async_copy(ones_v, acc_sh.at[dst_v.at[0]], sem).wait()

            pltpu.async_copy(ones_v, acc_sh.at[dst_v.at[j]], sem, add=True)
            return carry

        lax.fori_loop(0, TPW, body, 0)
        for _ in range(min(LAG, TPW)):
            pltpu.make_async_copy(ones_v, acc_sh.at[dst_v.at[0]], sem).wait()
        plsc.subcore_barrier()
        pltpu.sync_copy(acc_sh.at[pl.ds(s * RPT, RPT)], rb_v)
        pltpu.sync_copy(rb_v, out_hbm.at[c, pl.ds(s * RPT, RPT)])

    return deg_kernel(dst3, zvec)


def _agg_call(hs, src3, dst3, zrows):
    D = hs.shape[1]
    TPW = src3.shape[1]
    mesh = plsc.VectorSubcoreMesh(core_axis_name="c", subcore_axis_name="s")

    @functools.partial(
        pl.kernel,
        out_type=jax.ShapeDtypeStruct((NC, NP, D), jnp.float32),
        mesh=mesh,
        compiler_params=pltpu.CompilerParams(use_tc_tiling_on_sc=False),
        scratch_types=[
            pltpu.VMEM((TPW, L), jnp.int32),
            pltpu.VMEM((TPW, L), jnp.int32),
            pltpu.VMEM((RB * L, D), jnp.float32),
            pltpu.VMEM((RC, D), jnp.float32),
            pltpu.VMEM_SHARED((NP, D), jnp.float32),
            pltpu.SemaphoreType.DMA,
            pltpu.SemaphoreType.DMA,
        ],
    )
    def agg_kernel(hs_hbm, src_hbm, dst_hbm, zrows_hbm, out_hbm,
                   src_v, dst_v, rows_v, big_v, acc_sh, gsem, ssem):
        c = lax.axis_index("c")
        s = lax.axis_index("s")
        w = c * NS + s
        tpw = TPW
        pltpu.sync_copy(zrows_hbm, big_v)
        for h in range(RPT // RC):
            pltpu.sync_copy(big_v, acc_sh.at[pl.ds(s * RPT + h * RC, RC)])
        pltpu.sync_copy(src_hbm.at[w], src_v)
        pltpu.sync_copy(dst_hbm.at[w], dst_v)
        plsc.subcore_barrier()

        for b in range(RB - 1):
            pltpu.async_copy(hs_hbm.at[src_v.at[b]],
                             rows_v.at[pl.ds(b * L, L)], gsem)

        def body(j, carry):
            slot = lax.rem(j, RB) * L
            nxt = j + RB - 1
            nslot = lax.rem(nxt, RB) * L

            @pl.when(j >= 1)
            def _():
                pltpu.make_async_copy(rows_v.at[pl.ds(0, L)],
                                      acc_sh.at[dst_v.at[0]], ssem).wait()

            @pl.when(nxt < tpw)
            def _():
                pltpu.async_copy(hs_hbm.at[src_v.at[nxt]],
                                 rows_v.at[pl.ds(nslot, L)], gsem)

            pltpu.make_async_copy(hs_hbm.at[src_v.at[j]],
                                  rows_v.at[pl.ds(slot, L)], gsem).wait()
            pltpu.async_copy(rows_v.at[pl.ds(slot, L)],
                             acc_sh.at[dst_v.at[j]], ssem, add=True)
            return carry

        lax.fori_loop(0, tpw, body, 0)
        pltpu.make_async_copy(rows_v.at[pl.ds(0, L)],
                              acc_sh.at[dst_v.at[0]], ssem).wait()
        plsc.subcore_barrier()
        for h in range(RPT // RC):
            pltpu.sync_copy(acc_sh.at[pl.ds(s * RPT + h * RC, RC)], big_v)
            pltpu.sync_copy(big_v, out_hbm.at[c, pl.ds(s * RPT + h * RC, RC)])

    return agg_kernel(hs, src3, dst3, zrows)


def _hs_body(x_ref, w1_ref, w2_ref, degt_ref, out_ref):
    wc = lax.dot_general(w2_ref[...], w1_ref[...], (((1,), (0,)), ((), ())),
                         precision=_P, preferred_element_type=jnp.float32)
    h = lax.dot_general(x_ref[...], wc, (((1,), (1,)), ((), ())),
                        precision=_P, preferred_element_type=jnp.float32)
    deg = jnp.sum(degt_ref[...], axis=1, keepdims=True) + 1.0
    out_ref[...] = h * lax.rsqrt(deg)


def _final_body(p0_ref, p1_ref, hs_ref, degt_ref, w2_ref, b1_ref, b2_ref, out_ref):
    deg = jnp.sum(degt_ref[...], axis=1, keepdims=True) + 1.0
    dinv = lax.rsqrt(deg)
    bc = lax.dot_general(b1_ref[...], w2_ref[...], (((1,), (1,)), ((), ())),
                         precision=_P, preferred_element_type=jnp.float32)
    out_ref[...] = dinv * (p0_ref[...] + p1_ref[...] + hs_ref[...]) + (bc + b2_ref[...])


def kernel(x, edge_index, W1, b1, W2, b2):
    D = W2.shape[0]
    NF = x.shape[1]
    E = edge_index.shape[1]
    EPW = ((E + NW * L - 1) // (NW * L)) * L
    EP = EPW * NW
    TPW = EPW // L
    EW = E // NW
    PW = EPW - EW
    kk = jnp.arange(PW, dtype=jnp.int32)
    ww = jnp.arange(NW, dtype=jnp.int32)[:, None]
    pad_src = (ww * 331 + kk[None, :] * 13) % N
    pad_dst = N + (ww * 7 + kk[None, :]) % (NP - N)
    src3 = jnp.concatenate(
        [edge_index[0].reshape(NW, EW), pad_src], axis=1).reshape(NW, TPW, L)
    dst3 = jnp.concatenate(
        [edge_index[1].reshape(NW, EW), pad_dst], axis=1).reshape(NW, TPW, L)

    zvec = jnp.zeros((RPT,), jnp.float32)
    zrows = jnp.zeros((RC, D), jnp.float32)

    deg2 = _deg_call(dst3, zvec)
    deg_t = deg2.T

    NR = x.shape[0]
    NB = NR // 5

    hs = pl.pallas_call(
        _hs_body,
        grid=(NR // NB,),
        in_specs=[
            pl.BlockSpec((NB, NF), lambda i: (i, 0)),
            pl.BlockSpec(W1.shape, lambda i: (0, 0)),
            pl.BlockSpec(W2.shape, lambda i: (0, 0)),
            pl.BlockSpec((NB, NC), lambda i: (i, 0)),
        ],
        out_specs=pl.BlockSpec((NB, D), lambda i: (i, 0)),
        out_shape=jax.ShapeDtypeStruct((NR, D), jnp.float32),
    )(x, W1, W2, deg_t)

    parts = _agg_call(hs, src3, dst3, zrows)

    out = pl.pallas_call(
        _final_body,
        grid=(NR // NB,),
        in_specs=[
            pl.BlockSpec((NB, D), lambda i: (i, 0)),
            pl.BlockSpec((NB, D), lambda i: (i, 0)),
            pl.BlockSpec((NB, D), lambda i: (i, 0)),
            pl.BlockSpec((NB, NC), lambda i: (i, 0)),
            pl.BlockSpec(W2.shape, lambda i: (0, 0)),
            pl.BlockSpec((1, W1.shape[1]), lambda i: (0, 0)),
            pl.BlockSpec((1, D), lambda i: (0, 0)),
        ],
        out_specs=pl.BlockSpec((NB, D), lambda i: (i, 0)),
        out_shape=jax.ShapeDtypeStruct((NR, D), jnp.float32),
    )(parts[0], parts[1], hs, deg_t, W2, b1.reshape(1, -1), b2.reshape(1, -1))

    return out
